# baseline (speedup 1.0000x reference)
import jax
import jax.numpy as jnp
from jax import lax
from jax.experimental import pallas as pl

_JITTER_EPS = 0.1
_NEG_INF = float("-inf")


def _router_kernel(*refs):
    *x_refs, w_ref, gates_ref, mult_ref, sel_ref = refs
    n_split = len(x_refs)
    w = w_ref[...]
    ks = w.shape[1] // n_split
    scores = None
    for j, x_ref in enumerate(x_refs):
        part = lax.dot_general(
            x_ref[...],
            w[:, j * ks : (j + 1) * ks],
            (((1,), (1,)), ((), ())),
            preferred_element_type=jnp.float32,
        )
        scores = part if scores is None else scores + part

    t, e = scores.shape
    iota = lax.broadcasted_iota(jnp.int32, (t, e), 1)
    thr = 2.0 * _JITTER_EPS

    def argmin_at(eq_tile):
        return jnp.min(jnp.where(eq_tile, iota, e), axis=-1, keepdims=True)

    max_logit = jnp.max(scores, axis=-1, keepdims=True)
    max_ind = argmin_at(scores == max_logit)
    ex0 = jnp.exp(scores - max_logit)
    sum0 = jnp.sum(ex0, axis=-1, keepdims=True)
    gates_ref[...] = ex0 / sum0

    factor = jnp.maximum(jnp.abs(scores), max_logit)
    mask = (max_logit - scores) > thr * factor
    ex1 = jnp.where(mask, 0.0, ex0)
    sum1 = jnp.sum(ex1, axis=-1, keepdims=True)
    inv1 = 1.0 / sum1
    mg_max_ind = argmin_at(ex1 == 1.0)
    mask_for_one = 0.3333 + 0.6667 * (max_ind == mg_max_ind).astype(jnp.float32)
    mult1 = inv1 * mask_for_one

    is_sel = iota == max_ind
    ms = jnp.where(is_sel, _NEG_INF, scores)
    max_logit2 = jnp.max(ms, axis=-1, keepdims=True)
    max_ind2 = argmin_at(ms == max_logit2)
    factor2 = jnp.maximum(jnp.abs(scores), max_logit2)
    mask2 = (max_logit2 - scores) > thr * factor2
    ex2 = jnp.where(jnp.logical_or(mask2, is_sel), 0.0, jnp.exp(scores - max_logit2))
    sum2 = jnp.sum(ex2, axis=-1, keepdims=True)
    inv2 = 1.0 / sum2
    mg2_max_ind = argmin_at(ex2 == 1.0)
    mask_for_one2 = 0.3333 + 0.6667 * (max_ind2 == mg2_max_ind).astype(jnp.float32)
    mult2 = inv2 * mask_for_one2

    mult_ref[...] = jnp.concatenate([mult1, mult2], axis=-1)
    sel_ref[...] = jnp.concatenate([max_ind, max_ind2], axis=-1)


def kernel(x, W):
    n_tokens, d_model = x.shape
    n_experts = W.shape[0]
    t_blk = 1024
    n_split = 1
    ks = d_model // n_split
    grid = (n_tokens // t_blk,)
    gates, mult, sel = pl.pallas_call(
        _router_kernel,
        grid=grid,
        in_specs=[
            pl.BlockSpec((t_blk, ks), lambda i, _j=j: (i, _j))
            for j in range(n_split)
        ]
        + [
            pl.BlockSpec((n_experts, d_model), lambda i: (0, 0)),
        ],
        out_specs=[
            pl.BlockSpec((t_blk, n_experts), lambda i: (i, 0)),
            pl.BlockSpec((t_blk, 2), lambda i: (i, 0)),
            pl.BlockSpec((t_blk, 2), lambda i: (i, 0)),
        ],
        out_shape=[
            jax.ShapeDtypeStruct((n_tokens, n_experts), jnp.float32),
            jax.ShapeDtypeStruct((n_tokens, 2), jnp.float32),
            jax.ShapeDtypeStruct((n_tokens, 2), jnp.int32),
        ],
    )(*([x] * n_split), W)
    return mult, gates, sel


# speedup vs baseline: 1.0018x; 1.0018x over previous
import jax
import jax.numpy as jnp
from jax import lax
from jax.experimental import pallas as pl

_JITTER_EPS = 0.1
_NEG_INF = float("-inf")


def _router_kernel(*refs):
    *x_refs, w_ref, gates_ref, mult_ref, sel_ref = refs
    n_split = len(x_refs)
    w = w_ref[...]
    ks = w.shape[1] // n_split
    scores = None
    for j, x_ref in enumerate(x_refs):
        part = lax.dot_general(
            x_ref[...],
            w[:, j * ks : (j + 1) * ks],
            (((1,), (1,)), ((), ())),
            preferred_element_type=jnp.float32,
        )
        scores = part if scores is None else scores + part

    t, e = scores.shape
    iota = lax.broadcasted_iota(jnp.int32, (t, e), 1)
    thr = 2.0 * _JITTER_EPS

    def argmin_at(eq_tile):
        return jnp.min(jnp.where(eq_tile, iota, e), axis=-1, keepdims=True)

    max_logit = jnp.max(scores, axis=-1, keepdims=True)
    max_ind = argmin_at(scores == max_logit)
    ex0 = jnp.exp(scores - max_logit)
    sum0 = jnp.sum(ex0, axis=-1, keepdims=True)
    gates_ref[...] = ex0 / sum0

    factor = jnp.maximum(jnp.abs(scores), max_logit)
    mask = (max_logit - scores) > thr * factor
    ex1 = jnp.where(mask, 0.0, ex0)
    sum1 = jnp.sum(ex1, axis=-1, keepdims=True)
    inv1 = 1.0 / sum1
    mg_max_ind = argmin_at(ex1 == 1.0)
    mask_for_one = 0.3333 + 0.6667 * (max_ind == mg_max_ind).astype(jnp.float32)
    mult1 = inv1 * mask_for_one

    is_sel = iota == max_ind
    ms = jnp.where(is_sel, _NEG_INF, scores)
    max_logit2 = jnp.max(ms, axis=-1, keepdims=True)
    max_ind2 = argmin_at(ms == max_logit2)
    factor2 = jnp.maximum(jnp.abs(scores), max_logit2)
    mask2 = (max_logit2 - scores) > thr * factor2
    ex2 = jnp.where(jnp.logical_or(mask2, is_sel), 0.0, jnp.exp(scores - max_logit2))
    sum2 = jnp.sum(ex2, axis=-1, keepdims=True)
    inv2 = 1.0 / sum2
    mg2_max_ind = argmin_at(ex2 == 1.0)
    mask_for_one2 = 0.3333 + 0.6667 * (max_ind2 == mg2_max_ind).astype(jnp.float32)
    mult2 = inv2 * mask_for_one2

    mult_ref[...] = jnp.concatenate([mult1, mult2], axis=-1)
    sel_ref[...] = jnp.concatenate([max_ind, max_ind2], axis=-1)


def kernel(x, W):
    n_tokens, d_model = x.shape
    n_experts = W.shape[0]
    t_blk = 1024
    n_split = 2
    ks = d_model // n_split
    grid = (n_tokens // t_blk,)
    gates, mult, sel = pl.pallas_call(
        _router_kernel,
        grid=grid,
        in_specs=[
            pl.BlockSpec((t_blk, ks), lambda i, _j=j: (i, _j))
            for j in range(n_split)
        ]
        + [
            pl.BlockSpec((n_experts, d_model), lambda i: (0, 0)),
        ],
        out_specs=[
            pl.BlockSpec((t_blk, n_experts), lambda i: (i, 0)),
            pl.BlockSpec((t_blk, 2), lambda i: (i, 0)),
            pl.BlockSpec((t_blk, 2), lambda i: (i, 0)),
        ],
        out_shape=[
            jax.ShapeDtypeStruct((n_tokens, n_experts), jnp.float32),
            jax.ShapeDtypeStruct((n_tokens, 2), jnp.float32),
            jax.ShapeDtypeStruct((n_tokens, 2), jnp.int32),
        ],
    )(*([x] * n_split), W)
    return mult, gates, sel
